# baseline (speedup 1.0000x reference)
import functools

import jax
import jax.numpy as jnp
from jax import lax
from jax.experimental import pallas as pl
from jax.experimental.pallas import tpu as pltpu
from jax.experimental.pallas import tpu_sc as plsc

N_NODES = 10000
DIM = 128
NUM_REL = 2
NUM_BASES = 30
N_EDGES = 320000

STRIDE = 10240
HSIZE = NUM_REL * STRIDE
NE_PAD = 327680
EA = NE_PAD // 16
EB = NE_PAD // 32
NCA = EA // 1024
NCB = EB // 1024
RSL = HSIZE // 16


REAL_A_TILE = 15
REAL_A_LEN = N_EDGES - REAL_A_TILE * EA
REAL_B_WID = 31
REAL_B_LEN = N_EDGES - REAL_B_WID * EB
GAP = STRIDE - N_NODES


def _sc_edge_kernel(edges, etype, a_out,
                    e0, e1, esrc, hist, acc, tmp2d, rbuf, sem, sem2,
                    stage_sp):
    c = lax.axis_index("c")
    s = lax.axis_index("s")
    wid = s * 2 + c
    hb = c * EB

    one16 = jnp.full((16,), 1.0, jnp.float32)
    zero16 = jnp.zeros((16,), jnp.float32)

    @pl.when(wid != REAL_B_WID)
    def _fire_src_full():
        pltpu.async_copy(edges.at[0, pl.ds(wid * EB, EB)], esrc, sem2)

    @pl.when(wid == REAL_B_WID)
    def _fire_src_tail():
        pltpu.async_copy(edges.at[0, pl.ds(REAL_B_WID * EB, REAL_B_LEN)],
                         esrc.at[pl.ds(0, REAL_B_LEN)], sem2)

    base_a = s * EA

    @pl.when(s != REAL_A_TILE)
    def _load_a_full():
        pltpu.sync_copy(edges.at[1, pl.ds(base_a, EA)], e0)
        pltpu.sync_copy(etype.at[pl.ds(base_a, EA)], e1)

    @pl.when(s == REAL_A_TILE)
    def _load_a_tail():
        pltpu.sync_copy(edges.at[1, pl.ds(REAL_A_TILE * EA, REAL_A_LEN)],
                        e0.at[pl.ds(0, REAL_A_LEN)])
        pltpu.sync_copy(etype.at[pl.ds(REAL_A_TILE * EA, REAL_A_LEN)],
                        e1.at[pl.ds(0, REAL_A_LEN)])
        gap16 = N_NODES + lax.rem(lax.iota(jnp.int32, 16), GAP)
        zero16i = jnp.zeros((16,), jnp.int32)

        @plsc.parallel_loop(0, (EA - REAL_A_LEN) // 16)
        def _fill_fake_a(i):
            off = REAL_A_LEN + i * 16
            e0[pl.ds(off, 16)] = N_NODES + lax.rem(gap16 + i * 16 - N_NODES,
                                                   GAP)
            e1[pl.ds(off, 16)] = zero16i

    @plsc.parallel_loop(0, HSIZE // 64)
    def _zero_hist(i):
        for k in range(4):
            hist[pl.ds(i * 64 + k * 16, 16)] = zero16

    with jax.named_scope("hist_a"):
        @plsc.parallel_loop(0, EA // 64)
        def _hist_a(i):
            for k in range(4):
                o = i * 64 + k * 16
                d = e0[pl.ds(o, 16)]
                t = e1[pl.ds(o, 16)]
                plsc.addupdate_scatter(hist, [t * STRIDE + d], one16)

    HH = HSIZE // 2
    RH = RSL // 2
    scope_reduce = jax.named_scope("reduce_hist")
    scope_reduce.__enter__()

    for h in range(2):
        pltpu.sync_copy(hist.at[pl.ds(h * HH, HH)], stage_sp.at[s])
        plsc.subcore_barrier()
        pltpu.sync_copy(stage_sp.at[:, pl.ds(s * RH, RH)], tmp2d)

        @plsc.parallel_loop(0, RH // 16)
        def _reduce_r(i, h=h):
            v = tmp2d[0, pl.ds(i * 16, 16)]
            for r in range(1, 16):
                v = v + tmp2d[r, pl.ds(i * 16, 16)]
            rbuf[pl.ds(h * RH + i * 16, 16)] = 1.0 / jnp.maximum(v, 1.0)

        plsc.subcore_barrier()

    pltpu.sync_copy(rbuf.at[pl.ds(0, RH)], stage_sp.at[0, pl.ds(s * RH, RH)])
    pltpu.sync_copy(rbuf.at[pl.ds(RH, RH)], stage_sp.at[1, pl.ds(s * RH, RH)])
    plsc.subcore_barrier()
    pltpu.sync_copy(stage_sp.at[0], hist.at[pl.ds(0, HH)])
    pltpu.sync_copy(stage_sp.at[1], hist.at[pl.ds(HH, HH)])

    scope_reduce.__exit__(None, None, None)

    @plsc.parallel_loop(0, HSIZE // 64)
    def _zero_acc(i):
        for k in range(4):
            acc[pl.ds(i * 64 + k * 16, 16)] = zero16

    @pl.when(wid != REAL_B_WID)
    def _wait_src_full():
        pltpu.make_async_copy(edges.at[0, pl.ds(wid * EB, EB)], esrc,
                              sem2).wait()

    @pl.when(wid == REAL_B_WID)
    def _wait_src_tail():
        pltpu.make_async_copy(edges.at[0, pl.ds(REAL_B_WID * EB, REAL_B_LEN)],
                              esrc.at[pl.ds(0, REAL_B_LEN)], sem2).wait()
        gap16 = N_NODES + lax.rem(lax.iota(jnp.int32, 16), GAP)

        @plsc.parallel_loop(0, (EB - REAL_B_LEN) // 16)
        def _fill_fake_src(i):
            esrc[pl.ds(REAL_B_LEN + i * 16, 16)] = N_NODES + lax.rem(
                gap16 + i * 16 - N_NODES, GAP)

    with jax.named_scope("accum_b"):
        @plsc.parallel_loop(0, EB // 64)
        def _accum_b(i):
            for k in range(4):
                o = i * 64 + k * 16
                d = e0[pl.ds(hb + o, 16)]
                t = e1[pl.ds(hb + o, 16)]
                w = plsc.load_gather(hist, [t * STRIDE + d])
                sv = esrc[pl.ds(o, 16)]
                plsc.addupdate_scatter(acc, [t * STRIDE + sv], w)

    for h in range(2):
        plsc.subcore_barrier()
        pltpu.sync_copy(acc.at[pl.ds(h * HH, HH)], stage_sp.at[s])
        plsc.subcore_barrier()
        pltpu.sync_copy(stage_sp.at[:, pl.ds(s * RH, RH)], tmp2d)

        @plsc.parallel_loop(0, RH // 16)
        def _reduce_a(i):
            v = tmp2d[0, pl.ds(i * 16, 16)]
            for r in range(1, 16):
                v = v + tmp2d[r, pl.ds(i * 16, 16)]
            rbuf[pl.ds(i * 16, 16)] = v

        pltpu.sync_copy(rbuf.at[pl.ds(0, RH)],
                        a_out.at[c, pl.ds(h * HH + s * RH, RH)])


@functools.cache
def _sc_edge():
  return functools.partial(
    pl.kernel,
    out_type=jax.ShapeDtypeStruct((2, HSIZE), jnp.float32),
    mesh=plsc.VectorSubcoreMesh(core_axis_name="c", subcore_axis_name="s"),
    compiler_params=pltpu.CompilerParams(needs_layout_passes=False),
    scratch_types=[
        pltpu.VMEM((EA,), jnp.int32),
        pltpu.VMEM((EA,), jnp.int32),
        pltpu.VMEM((EB,), jnp.int32),
        pltpu.VMEM((HSIZE,), jnp.float32),
        pltpu.VMEM((HSIZE,), jnp.float32),
        pltpu.VMEM((16, RSL // 2), jnp.float32),
        pltpu.VMEM((RSL,), jnp.float32),
        pltpu.SemaphoreType.DMA,
        pltpu.SemaphoreType.DMA,
        pltpu.VMEM_SHARED((16, HSIZE // 2), jnp.float32),
    ],
  )(_sc_edge_kernel)


NBLK = 2048
NGRID = (N_NODES + NBLK - 1) // NBLK


def _tc_base_kernel(x_ref, root_ref, bias_ref, linW_ref, linb_ref, out_ref):
    cs = jnp.sum(x_ref[...], axis=0, keepdims=True)
    rootp = jnp.dot(cs, root_ref[...], preferred_element_type=jnp.float32)
    pooled = rootp * (1.0 / N_NODES) + bias_ref[...]
    out_ref[...] = jnp.dot(pooled, linW_ref[...],
                           preferred_element_type=jnp.float32) + linb_ref[...]


_tc_base = pl.pallas_call(
    _tc_base_kernel,
    out_shape=jax.ShapeDtypeStruct((1, N_NODES), jnp.float32),
)


def _tc_delta_kernel(apart_ref, x_ref, compT_ref, basis_ref, out_ref):
    asum = apart_ref[0, :] + apart_ref[1, :]
    a0 = asum[0:N_NODES][None, :]
    a1 = asum[STRIDE:STRIDE + N_NODES][None, :]
    m = jnp.concatenate([a0, a1], axis=0)
    msum = jnp.dot(m, x_ref[...], preferred_element_type=jnp.float32)
    p = jnp.dot(compT_ref[...], msum, preferred_element_type=jnp.float32)
    acc = jnp.zeros((1, DIM), jnp.float32)
    for b in range(NUM_BASES):
        acc = acc + jnp.dot(p[b:b + 1, :], basis_ref[b],
                            preferred_element_type=jnp.float32)
    out_ref[...] = acc * (1.0 / N_NODES)


_tc_delta = pl.pallas_call(
    _tc_delta_kernel,
    out_shape=jax.ShapeDtypeStruct((1, DIM), jnp.float32),
)


def _tc_final_kernel(delta_ref, linW_ref, base_ref, out_ref):
    out_ref[...] = base_ref[...] + jnp.dot(delta_ref[...], linW_ref[...],
                                           preferred_element_type=jnp.float32)


_tc_final = pl.pallas_call(
    _tc_final_kernel,
    grid=(NGRID,),
    in_specs=[
        pl.BlockSpec((1, DIM), lambda i: (0, 0)),
        pl.BlockSpec((DIM, NBLK), lambda i: (0, i)),
        pl.BlockSpec((1, NBLK), lambda i: (0, i)),
    ],
    out_specs=pl.BlockSpec((1, NBLK), lambda i: (0, i)),
    out_shape=jax.ShapeDtypeStruct((1, N_NODES), jnp.float32),
)


@jax.jit
def kernel(edges, edge_type, item_emb, basis, comp, root, rgcn_bias, lin_W, lin_b):
    apart = _sc_edge()(edges.astype(jnp.int32), edge_type.astype(jnp.int32))
    base = _tc_base(item_emb, root, rgcn_bias[None, :], lin_W, lin_b[None, :])
    delta = _tc_delta(apart, item_emb, comp.T, basis)
    return _tc_final(delta, lin_W, base)


# speedup vs baseline: 1.1942x; 1.1942x over previous
import functools

import jax
import jax.numpy as jnp
from jax import lax
from jax.experimental import pallas as pl
from jax.experimental.pallas import tpu as pltpu
from jax.experimental.pallas import tpu_sc as plsc

N_NODES = 10000
DIM = 128
NUM_REL = 2
NUM_BASES = 30
N_EDGES = 320000

STRIDE = 10240
HSIZE = NUM_REL * STRIDE
NE_PAD = 327680
EA = NE_PAD // 16
EB = NE_PAD // 32
NCA = EA // 1024
NCB = EB // 1024
RSL = HSIZE // 16


REAL_A_TILE = 15
REAL_A_LEN = N_EDGES - REAL_A_TILE * EA
REAL_B_WID = 31
REAL_B_LEN = N_EDGES - REAL_B_WID * EB
GAP = STRIDE - N_NODES


def _sc_edge_kernel(edges, etype, a_out,
                    e0, e1, esrc, hist, acc, tmp2d, rbuf, sem, sem2,
                    stage_sp):
    c = lax.axis_index("c")
    s = lax.axis_index("s")
    wid = s * 2 + c
    hb = c * EB

    one16 = jnp.full((16,), 1.0, jnp.float32)
    zero16 = jnp.zeros((16,), jnp.float32)

    @pl.when(wid != REAL_B_WID)
    def _fire_src_full():
        pltpu.async_copy(edges.at[0, pl.ds(wid * EB, EB)], esrc, sem2)

    @pl.when(wid == REAL_B_WID)
    def _fire_src_tail():
        pltpu.async_copy(edges.at[0, pl.ds(REAL_B_WID * EB, REAL_B_LEN)],
                         esrc.at[pl.ds(0, REAL_B_LEN)], sem2)

    base_a = s * EA

    @pl.when(s != REAL_A_TILE)
    def _fire_a_full():
        pltpu.async_copy(edges.at[1, pl.ds(base_a, EA)], e0, sem)
        pltpu.async_copy(etype.at[pl.ds(base_a, EA)], e1, sem)

    @pl.when(s == REAL_A_TILE)
    def _fire_a_tail():
        pltpu.async_copy(edges.at[1, pl.ds(REAL_A_TILE * EA, REAL_A_LEN)],
                         e0.at[pl.ds(0, REAL_A_LEN)], sem)
        pltpu.async_copy(etype.at[pl.ds(REAL_A_TILE * EA, REAL_A_LEN)],
                         e1.at[pl.ds(0, REAL_A_LEN)], sem)

    @plsc.parallel_loop(0, HSIZE // 64)
    def _zero_hist(i):
        for k in range(4):
            hist[pl.ds(i * 64 + k * 16, 16)] = zero16

    @plsc.parallel_loop(0, HSIZE // 64)
    def _zero_acc(i):
        for k in range(4):
            acc[pl.ds(i * 64 + k * 16, 16)] = zero16

    @pl.when(s != REAL_A_TILE)
    def _wait_a_full():
        pltpu.make_async_copy(edges.at[1, pl.ds(base_a, EA)], e0, sem).wait()
        pltpu.make_async_copy(etype.at[pl.ds(base_a, EA)], e1, sem).wait()

    @pl.when(s == REAL_A_TILE)
    def _wait_a_tail():
        pltpu.make_async_copy(edges.at[1, pl.ds(REAL_A_TILE * EA, REAL_A_LEN)],
                              e0.at[pl.ds(0, REAL_A_LEN)], sem).wait()
        pltpu.make_async_copy(etype.at[pl.ds(REAL_A_TILE * EA, REAL_A_LEN)],
                              e1.at[pl.ds(0, REAL_A_LEN)], sem).wait()
        gap16 = N_NODES + lax.rem(lax.iota(jnp.int32, 16), GAP)
        zero16i = jnp.zeros((16,), jnp.int32)

        @plsc.parallel_loop(0, (EA - REAL_A_LEN) // 16)
        def _fill_fake_a(i):
            off = REAL_A_LEN + i * 16
            e0[pl.ds(off, 16)] = N_NODES + lax.rem(gap16 + i * 16 - N_NODES,
                                                   GAP)
            e1[pl.ds(off, 16)] = zero16i

    with jax.named_scope("hist_a"):
        @plsc.parallel_loop(0, EA // 64)
        def _hist_a(i):
            for k in range(4):
                o = i * 64 + k * 16
                d = e0[pl.ds(o, 16)]
                t = e1[pl.ds(o, 16)]
                plsc.addupdate_scatter(hist, [t * STRIDE + d], one16)

    HH = HSIZE // 2
    RH = RSL // 2
    scope_reduce = jax.named_scope("reduce_hist")
    scope_reduce.__enter__()

    for h in range(2):
        pltpu.sync_copy(hist.at[pl.ds(h * HH, HH)], stage_sp.at[s])
        plsc.subcore_barrier()
        pltpu.sync_copy(stage_sp.at[:, pl.ds(s * RH, RH)], tmp2d)

        @plsc.parallel_loop(0, RH // 16)
        def _reduce_r(i, h=h):
            v = tmp2d[0, pl.ds(i * 16, 16)]
            for r in range(1, 16):
                v = v + tmp2d[r, pl.ds(i * 16, 16)]
            rbuf[pl.ds(h * RH + i * 16, 16)] = 1.0 / jnp.maximum(v, 1.0)

        plsc.subcore_barrier()

    pltpu.sync_copy(rbuf.at[pl.ds(0, RH)], stage_sp.at[0, pl.ds(s * RH, RH)])
    pltpu.sync_copy(rbuf.at[pl.ds(RH, RH)], stage_sp.at[1, pl.ds(s * RH, RH)])
    plsc.subcore_barrier()
    pltpu.sync_copy(stage_sp.at[0], hist.at[pl.ds(0, HH)])
    pltpu.sync_copy(stage_sp.at[1], hist.at[pl.ds(HH, HH)])

    scope_reduce.__exit__(None, None, None)

    @pl.when(wid != REAL_B_WID)
    def _wait_src_full():
        pltpu.make_async_copy(edges.at[0, pl.ds(wid * EB, EB)], esrc,
                              sem2).wait()

    @pl.when(wid == REAL_B_WID)
    def _wait_src_tail():
        pltpu.make_async_copy(edges.at[0, pl.ds(REAL_B_WID * EB, REAL_B_LEN)],
                              esrc.at[pl.ds(0, REAL_B_LEN)], sem2).wait()
        gap16 = N_NODES + lax.rem(lax.iota(jnp.int32, 16), GAP)

        @plsc.parallel_loop(0, (EB - REAL_B_LEN) // 16)
        def _fill_fake_src(i):
            esrc[pl.ds(REAL_B_LEN + i * 16, 16)] = N_NODES + lax.rem(
                gap16 + i * 16 - N_NODES, GAP)

    with jax.named_scope("accum_b"):
        @plsc.parallel_loop(0, EB // 64)
        def _accum_b(i):
            for k in range(4):
                o = i * 64 + k * 16
                d = e0[pl.ds(hb + o, 16)]
                t = e1[pl.ds(hb + o, 16)]
                w = plsc.load_gather(hist, [t * STRIDE + d])
                sv = esrc[pl.ds(o, 16)]
                plsc.addupdate_scatter(acc, [t * STRIDE + sv], w)

    with jax.named_scope("emit_acc"):
        pltpu.sync_copy(acc, a_out.at[wid])


@functools.cache
def _sc_edge():
  return functools.partial(
    pl.kernel,
    out_type=jax.ShapeDtypeStruct((32, HSIZE), jnp.float32),
    mesh=plsc.VectorSubcoreMesh(core_axis_name="c", subcore_axis_name="s"),
    compiler_params=pltpu.CompilerParams(needs_layout_passes=False),
    scratch_types=[
        pltpu.VMEM((EA,), jnp.int32),
        pltpu.VMEM((EA,), jnp.int32),
        pltpu.VMEM((EB,), jnp.int32),
        pltpu.VMEM((HSIZE,), jnp.float32),
        pltpu.VMEM((HSIZE,), jnp.float32),
        pltpu.VMEM((16, RSL // 2), jnp.float32),
        pltpu.VMEM((RSL,), jnp.float32),
        pltpu.SemaphoreType.DMA,
        pltpu.SemaphoreType.DMA,
        pltpu.VMEM_SHARED((16, HSIZE // 2), jnp.float32),
    ],
  )(_sc_edge_kernel)


NBLK = 2048
NGRID = (N_NODES + NBLK - 1) // NBLK


def _tc_base_kernel(x_ref, root_ref, bias_ref, linW_ref, linb_ref, out_ref):
    cs = jnp.sum(x_ref[...], axis=0, keepdims=True)
    rootp = jnp.dot(cs, root_ref[...], preferred_element_type=jnp.float32)
    pooled = rootp * (1.0 / N_NODES) + bias_ref[...]
    out_ref[...] = jnp.dot(pooled, linW_ref[...],
                           preferred_element_type=jnp.float32) + linb_ref[...]


_tc_base = pl.pallas_call(
    _tc_base_kernel,
    out_shape=jax.ShapeDtypeStruct((1, N_NODES), jnp.float32),
)


def _tc_main_kernel(apart_ref, x_ref, compT_ref, basis_ref, linW_ref,
                    base_ref, out_ref):
    asum = jnp.sum(apart_ref[...], axis=0)
    a0 = asum[0:N_NODES][None, :]
    a1 = asum[STRIDE:STRIDE + N_NODES][None, :]
    m = jnp.concatenate([a0, a1], axis=0)
    msum = jnp.dot(m, x_ref[...], preferred_element_type=jnp.float32)
    p = jnp.dot(compT_ref[...], msum, preferred_element_type=jnp.float32)
    acc = jnp.zeros((1, DIM), jnp.float32)
    for b in range(NUM_BASES):
        acc = acc + jnp.dot(p[b:b + 1, :], basis_ref[b],
                            preferred_element_type=jnp.float32)
    delta = acc * (1.0 / N_NODES)
    out_ref[...] = base_ref[...] + jnp.dot(delta, linW_ref[...],
                                           preferred_element_type=jnp.float32)


_tc_main = pl.pallas_call(
    _tc_main_kernel,
    out_shape=jax.ShapeDtypeStruct((1, N_NODES), jnp.float32),
)


@jax.jit
def kernel(edges, edge_type, item_emb, basis, comp, root, rgcn_bias, lin_W, lin_b):
    apart = _sc_edge()(edges.astype(jnp.int32), edge_type.astype(jnp.int32))
    base = _tc_base(item_emb, root, rgcn_bias[None, :], lin_W, lin_b[None, :])
    return _tc_main(apart, item_emb, comp.T, basis, lin_W, base)
